# baseline (speedup 1.0000x reference)
import jax
import jax.numpy as jnp
from jax import lax
from jax.experimental import pallas as pl
from jax.experimental.pallas import tpu as pltpu
from jax.experimental.pallas import tpu_sc as plsc

_D = 50
_G = 16
_GPL = 4
_NC = 2
_NS = 16
_NW = _NC * _NS
_CL = 32


def _embed_body(idx_hbm, table_hbm, out_hbm, idx_v, list_v, staged, outb, sem):
    per_w = idx_v.shape[0]
    nchunk = per_w // _CL
    wid = lax.axis_index("s") * _NC + lax.axis_index("c")
    base = wid * per_w
    pltpu.sync_copy(idx_hbm.at[wid], idx_v)

    lane = lax.iota(jnp.int32, 16)
    pos4 = lane * _GPL
    row4 = lane * _GPL
    dst50 = lane * _D

    @pl.loop(0, nchunk)
    def _chunk(c):
        i0 = c * _CL
        for h in range(2):
            r = idx_v[pl.ds(i0 + h * 16, 16)]
            g0 = lax.shift_right_logical(r * _D, 4)
            for k in range(_GPL):
                plsc.store_scatter(list_v, [pos4 + (64 * h + k)], g0 + k)
        pltpu.async_copy(table_hbm.at[list_v], staged, sem).wait()

        for h in range(2):
            r = idx_v[pl.ds(i0 + h * 16, 16)]
            o = jnp.bitwise_and(r * _D, _G - 1)
            src0 = row4 * _G + o + (64 * _G * h)
            dst0 = dst50 + (_D * 16 * h)

            @pl.loop(0, _D)
            def _j(j):
                src = src0 + j
                vals = plsc.load_gather(
                    staged,
                    [lax.shift_right_logical(src, 4),
                     jnp.bitwise_and(src, _G - 1)],
                )
                plsc.store_scatter(outb, [dst0 + j], vals)

        pltpu.sync_copy(outb, out_hbm.at[pl.ds((base + i0) * _D, _CL * _D)])


def kernel(indices, table):
    b, p = indices.shape
    total = b * p
    per_w = total // _NW
    idx = indices.reshape(_NW, per_w).astype(jnp.int32)
    tab_g = table.reshape(-1, _G)
    mesh = plsc.VectorSubcoreMesh(core_axis_name="c", subcore_axis_name="s")
    out = pl.kernel(
        _embed_body,
        out_type=jax.ShapeDtypeStruct((total * _D,), jnp.float32),
        mesh=mesh,
        scratch_types=[
            pltpu.VMEM((per_w,), jnp.int32),
            pltpu.VMEM((_GPL * _CL,), jnp.int32),
            pltpu.VMEM((_GPL * _CL, _G), jnp.float32),
            pltpu.VMEM((_CL * _D,), jnp.float32),
            pltpu.SemaphoreType.DMA,
        ],
        compiler_params=pltpu.CompilerParams(
            use_tc_tiling_on_sc=False, needs_layout_passes=False
        ),
    )(idx, tab_g)
    return out.reshape(b, p, _D)


# speedup vs baseline: 1.1295x; 1.1295x over previous
import jax
import jax.numpy as jnp
from jax import lax
from jax.experimental import pallas as pl
from jax.experimental.pallas import tpu as pltpu
from jax.experimental.pallas import tpu_sc as plsc

_D = 50
_G = 16
_GPL = 4
_NC = 2
_NS = 16
_NW = _NC * _NS
_CL = 32


def _embed_body(idx_hbm, table_hbm, out_hbm, idx_v,
                list0, list1, staged0, staged1, outb0, outb1,
                sem0, sem1, osem0, osem1):
    per_w = idx_v.shape[0]
    nchunk = per_w // _CL
    wid = lax.axis_index("s") * _NC + lax.axis_index("c")
    base = wid * per_w
    pltpu.sync_copy(idx_hbm.at[wid], idx_v)

    lists = (list0, list1)
    stageds = (staged0, staged1)
    outbs = (outb0, outb1)
    sems = (sem0, sem1)
    osems = (osem0, osem1)

    lane = lax.iota(jnp.int32, 16)
    pos4 = lane * _GPL
    dst50 = lane * _D

    def build_list(c, par):
        for h in range(2):
            r = idx_v[pl.ds(c * _CL + h * 16, 16)]
            g0 = lax.shift_right_logical(r * _D, 4)
            for k in range(_GPL):
                plsc.store_scatter(lists[par], [pos4 + (64 * h + k)], g0 + k)

    def fire_gather(par):
        pltpu.async_copy(table_hbm.at[lists[par]], stageds[par], sems[par])

    def wait_gather(par):
        pltpu.make_async_copy(
            table_hbm.at[lists[par]], stageds[par], sems[par]).wait()

    def extract(c, par):
        for h in range(2):
            r = idx_v[pl.ds(c * _CL + h * 16, 16)]
            o = jnp.bitwise_and(r * _D, _G - 1)
            src0 = lane * (_GPL * _G) + o + (64 * _G * h)
            dst0 = dst50 + (_D * 16 * h)

            @pl.loop(0, _D)
            def _j(j):
                src = src0 + j
                vals = plsc.load_gather(
                    stageds[par],
                    [lax.shift_right_logical(src, 4),
                     jnp.bitwise_and(src, _G - 1)],
                )
                plsc.store_scatter(outbs[par], [dst0 + j], vals)

    def out_slice(c):
        return out_hbm.at[pl.ds((base + c * _CL) * _D, _CL * _D)]

    def fire_out(c, par):
        pltpu.async_copy(outbs[par], out_slice(c), osems[par])

    def wait_out(par):
        pltpu.make_async_copy(outbs[par], out_slice(0), osems[par]).wait()

    build_list(0, 0)
    fire_gather(0)

    @pl.loop(0, nchunk // 2)
    def _cc(cc):
        for par in range(2):
            c = cc * 2 + par
            nxt = 1 - par

            @pl.when(c + 1 < nchunk)
            def _():
                build_list(c + 1, nxt)
                fire_gather(nxt)

            wait_gather(par)

            @pl.when(c >= 2)
            def _():
                wait_out(par)

            extract(c, par)
            fire_out(c, par)

    for par in range(2):
        wait_out(par)


def kernel(indices, table):
    b, p = indices.shape
    total = b * p
    per_w = total // _NW
    idx = indices.reshape(_NW, per_w).astype(jnp.int32)
    tab_g = table.reshape(-1, _G)
    mesh = plsc.VectorSubcoreMesh(core_axis_name="c", subcore_axis_name="s")
    out = pl.kernel(
        _embed_body,
        out_type=jax.ShapeDtypeStruct((total * _D,), jnp.float32),
        mesh=mesh,
        scratch_types=[
            pltpu.VMEM((per_w,), jnp.int32),
            pltpu.VMEM((_GPL * _CL,), jnp.int32),
            pltpu.VMEM((_GPL * _CL,), jnp.int32),
            pltpu.VMEM((_GPL * _CL, _G), jnp.float32),
            pltpu.VMEM((_GPL * _CL, _G), jnp.float32),
            pltpu.VMEM((_CL * _D,), jnp.float32),
            pltpu.VMEM((_CL * _D,), jnp.float32),
            pltpu.SemaphoreType.DMA,
            pltpu.SemaphoreType.DMA,
            pltpu.SemaphoreType.DMA,
            pltpu.SemaphoreType.DMA,
        ],
        compiler_params=pltpu.CompilerParams(
            use_tc_tiling_on_sc=False, needs_layout_passes=False
        ),
    )(idx, tab_g)
    return out.reshape(b, p, _D)


# speedup vs baseline: 1.2145x; 1.0752x over previous
import jax
import jax.numpy as jnp
from jax import lax
from jax.experimental import pallas as pl
from jax.experimental.pallas import tpu as pltpu
from jax.experimental.pallas import tpu_sc as plsc

_D = 50
_NC = 2
_NS = 16
_NW = _NC * _NS
_CL = 32
_COLS = (0, 16, 32, 34)


def _embed_body(idx_hbm, table_hbm, out_hbm,
                idx_v, idx_s, slots, outb, gsem0, gsem1, osem0, osem1):
    per_w = idx_v.shape[0]
    nchunk = per_w // _CL
    wid = lax.axis_index("s") * _NC + lax.axis_index("c")
    base = wid * per_w
    pltpu.sync_copy(idx_hbm.at[wid], idx_v)

    gsems = (gsem0, gsem1)
    osems = (osem0, osem1)

    lane = lax.iota(jnp.int32, 16)
    zeros16 = jnp.full((16,), 0, jnp.int32)

    def fire_chunk(c, par):
        for h in range(2):
            r_vec = idx_v[pl.ds(c * _CL + h * 16, 16)]
            for l in range(16):
                r = lax.reduce_sum(
                    jnp.where(lane == l, r_vec, zeros16), axes=(0,))
                idx_s[par, h * 16 + l] = r
                t = lax.shift_right_logical(r, 3)
                pltpu.async_copy(table_hbm.at[t],
                                 slots.at[par, h * 16 + l], gsems[par])

    def wait_chunk(par):
        for i in range(_CL):
            pltpu.make_async_copy(
                table_hbm.at[0], slots.at[par, i], gsems[par]).wait()

    def extract(par):
        for k in range(_CL):
            s = jnp.bitwise_and(idx_s[par, k], 7)
            for c0 in _COLS:
                outb[par, k, pl.ds(c0, 16)] = slots[par, k, s, pl.ds(c0, 16)]

    def out_slice(c):
        return out_hbm.at[pl.ds(base + c * _CL, _CL)]

    def fire_out(c, par):
        pltpu.async_copy(outb.at[par], out_slice(c), osems[par])

    def wait_out(par):
        pltpu.make_async_copy(outb.at[par], out_slice(0), osems[par]).wait()

    fire_chunk(0, 0)

    @pl.loop(0, nchunk // 2)
    def _cc(cc):
        for par in range(2):
            c = cc * 2 + par
            nxt = 1 - par

            @pl.when(c + 1 < nchunk)
            def _():
                fire_chunk(c + 1, nxt)

            wait_chunk(par)

            @pl.when(c >= 2)
            def _():
                wait_out(par)

            extract(par)
            fire_out(c, par)

    for par in range(2):
        wait_out(par)


def kernel(indices, table):
    b, p = indices.shape
    total = b * p
    per_w = total // _NW
    idx = indices.reshape(_NW, per_w).astype(jnp.int32)
    tab3 = table.reshape(table.shape[0] // 8, 8, _D)
    mesh = plsc.VectorSubcoreMesh(core_axis_name="c", subcore_axis_name="s")
    out = pl.kernel(
        _embed_body,
        out_type=jax.ShapeDtypeStruct((total, _D), jnp.float32),
        mesh=mesh,
        scratch_types=[
            pltpu.VMEM((per_w,), jnp.int32),
            pltpu.SMEM((2, _CL), jnp.int32),
            pltpu.VMEM((2, _CL, 8, _D), jnp.float32),
            pltpu.VMEM((2, _CL, _D), jnp.float32),
            pltpu.SemaphoreType.DMA,
            pltpu.SemaphoreType.DMA,
            pltpu.SemaphoreType.DMA,
            pltpu.SemaphoreType.DMA,
        ],
        compiler_params=pltpu.CompilerParams(
            use_tc_tiling_on_sc=True, needs_layout_passes=False
        ),
    )(idx, tab3)
    return out.reshape(b, p, _D)


# speedup vs baseline: 2.5786x; 2.1232x over previous
import jax
import jax.numpy as jnp
from jax import lax
from jax.experimental import pallas as pl
from jax.experimental.pallas import tpu as pltpu
from jax.experimental.pallas import tpu_sc as plsc

_D = 50
_NC = 2
_NS = 16
_NW = _NC * _NS
_CL = 32


def _embed_body(idx_hbm, table_hbm, out_hbm, idx_v, outb,
                gsem0, gsem1, osem0, osem1):
    per_w = idx_v.shape[0]
    nchunk = per_w // _CL
    wid = lax.axis_index("s") * _NC + lax.axis_index("c")
    base = wid * per_w
    pltpu.sync_copy(idx_hbm.at[wid], idx_v)

    gsems = (gsem0, gsem1)
    osems = (osem0, osem1)

    lane = lax.iota(jnp.int32, 16)
    zeros16 = jnp.full((16,), 0, jnp.int32)

    def fire_chunk(c, par):
        for h in range(2):
            r_vec = idx_v[pl.ds(c * _CL + h * 16, 16)]
            for l in range(16):
                r = lax.reduce_sum(
                    jnp.where(lane == l, r_vec, zeros16), axes=(0,))
                pltpu.async_copy(table_hbm.at[r],
                                 outb.at[par, h * 16 + l], gsems[par])

    def wait_chunk(par):
        for i in range(_CL):
            pltpu.make_async_copy(
                table_hbm.at[0], outb.at[par, i], gsems[par]).wait()

    def out_slice(c):
        return out_hbm.at[pl.ds(base + c * _CL, _CL)]

    def fire_out(c, par):
        pltpu.async_copy(outb.at[par], out_slice(c), osems[par])

    def wait_out(par):
        pltpu.make_async_copy(outb.at[par], out_slice(0), osems[par]).wait()

    fire_chunk(0, 0)

    @pl.loop(0, nchunk // 2)
    def _cc(cc):
        for par in range(2):
            c = cc * 2 + par
            nxt = 1 - par

            @pl.when(c + 1 < nchunk)
            def _():
                fire_chunk(c + 1, nxt)

            wait_chunk(par)

            @pl.when(c >= 2)
            def _():
                wait_out(par)

            fire_out(c, par)

    for par in range(2):
        wait_out(par)


def kernel(indices, table):
    b, p = indices.shape
    total = b * p
    per_w = total // _NW
    idx = indices.reshape(_NW, per_w).astype(jnp.int32)
    mesh = plsc.VectorSubcoreMesh(core_axis_name="c", subcore_axis_name="s")
    out = pl.kernel(
        _embed_body,
        out_type=jax.ShapeDtypeStruct((total, _D), jnp.float32),
        mesh=mesh,
        scratch_types=[
            pltpu.VMEM((per_w,), jnp.int32),
            pltpu.VMEM((2, _CL, _D), jnp.float32),
            pltpu.SemaphoreType.DMA,
            pltpu.SemaphoreType.DMA,
            pltpu.SemaphoreType.DMA,
            pltpu.SemaphoreType.DMA,
        ],
        compiler_params=pltpu.CompilerParams(
            use_tc_tiling_on_sc=True, needs_layout_passes=False
        ),
    )(idx, table)
    return out.reshape(b, p, _D)


# speedup vs baseline: 2.5858x; 1.0028x over previous
import jax
import jax.numpy as jnp
from jax import lax
from jax.experimental import pallas as pl
from jax.experimental.pallas import tpu as pltpu
from jax.experimental.pallas import tpu_sc as plsc

_D = 50
_NC = 2
_NS = 16
_NW = _NC * _NS
_CL = 32


def _embed_body(idx_hbm, table_hbm, out_hbm, idx_v, outb,
                gsem0, gsem1, osem0, osem1):
    per_w = idx_v.shape[0]
    nchunk = per_w // _CL
    wid = lax.axis_index("s") * _NC + lax.axis_index("c")
    base = wid * per_w
    pltpu.sync_copy(idx_hbm.at[wid], idx_v)

    gsems = (gsem0, gsem1)
    osems = (osem0, osem1)

    lane = lax.iota(jnp.int32, 16)
    zeros16 = jnp.full((16,), 0, jnp.int32)

    def fire_chunk(c, par):
        for h in range(2):
            r_vec = idx_v[pl.ds(c * _CL + h * 16, 16)]
            for l in range(16):
                r = r_vec[l]
                pltpu.async_copy(table_hbm.at[r],
                                 outb.at[par, h * 16 + l], gsems[par])

    def wait_chunk(par):
        for i in range(_CL):
            pltpu.make_async_copy(
                table_hbm.at[0], outb.at[par, i], gsems[par]).wait()

    def out_slice(c):
        return out_hbm.at[pl.ds(base + c * _CL, _CL)]

    def fire_out(c, par):
        pltpu.async_copy(outb.at[par], out_slice(c), osems[par])

    def wait_out(par):
        pltpu.make_async_copy(outb.at[par], out_slice(0), osems[par]).wait()

    fire_chunk(0, 0)

    @pl.loop(0, nchunk // 2)
    def _cc(cc):
        for par in range(2):
            c = cc * 2 + par
            nxt = 1 - par

            @pl.when(c + 1 < nchunk)
            def _():
                fire_chunk(c + 1, nxt)

            wait_chunk(par)

            @pl.when(c >= 2)
            def _():
                wait_out(par)

            fire_out(c, par)

    for par in range(2):
        wait_out(par)


def kernel(indices, table):
    b, p = indices.shape
    total = b * p
    per_w = total // _NW
    idx = indices.reshape(_NW, per_w).astype(jnp.int32)
    mesh = plsc.VectorSubcoreMesh(core_axis_name="c", subcore_axis_name="s")
    out = pl.kernel(
        _embed_body,
        out_type=jax.ShapeDtypeStruct((total, _D), jnp.float32),
        mesh=mesh,
        scratch_types=[
            pltpu.VMEM((per_w,), jnp.int32),
            pltpu.VMEM((2, _CL, _D), jnp.float32),
            pltpu.SemaphoreType.DMA,
            pltpu.SemaphoreType.DMA,
            pltpu.SemaphoreType.DMA,
            pltpu.SemaphoreType.DMA,
        ],
        compiler_params=pltpu.CompilerParams(
            use_tc_tiling_on_sc=True, needs_layout_passes=False
        ),
    )(idx, table)
    return out.reshape(b, p, _D)


# speedup vs baseline: 3.1681x; 1.2252x over previous
import jax
import jax.numpy as jnp
from jax import lax
from jax.experimental import pallas as pl
from jax.experimental.pallas import tpu as pltpu
from jax.experimental.pallas import tpu_sc as plsc

_D = 50
_NC = 2
_NS = 16
_NW = _NC * _NS
_BC = 2


def _embed_body(idx_hbm, table_hbm, out_hbm, idx_v, outb,
                gsem0, gsem1, osem0, osem1):
    per_w, p = idx_v.shape
    cl = _BC * p
    nchunk = per_w // _BC
    wid = lax.axis_index("s") * _NC + lax.axis_index("c")
    base_b = wid * per_w
    pltpu.sync_copy(idx_hbm.at[pl.ds(base_b, per_w)], idx_v)

    gsems = (gsem0, gsem1)
    osems = (osem0, osem1)

    def fire_chunk(c, par):
        for i in range(_BC):
            bb = c * _BC + i
            v0 = idx_v[bb, pl.ds(0, 16)]
            v1 = idx_v[bb, pl.ds(p - 16, 16)]
            for k in range(16):
                pltpu.async_copy(table_hbm.at[v0[k]],
                                 outb.at[par, i, k], gsems[par])
            for k in range(16, p):
                pltpu.async_copy(table_hbm.at[v1[k - (p - 16)]],
                                 outb.at[par, i, k], gsems[par])

    def wait_chunk(par):
        for i in range(_BC):
            for j in range(p):
                pltpu.make_async_copy(
                    table_hbm.at[0], outb.at[par, i, j], gsems[par]).wait()

    def out_slice(c):
        return out_hbm.at[pl.ds(base_b + c * _BC, _BC)]

    def fire_out(c, par):
        pltpu.async_copy(outb.at[par], out_slice(c), osems[par])

    def wait_out(par):
        pltpu.make_async_copy(outb.at[par], out_slice(0), osems[par]).wait()

    fire_chunk(0, 0)

    @pl.loop(0, nchunk // 2)
    def _cc(cc):
        for par in range(2):
            c = cc * 2 + par
            nxt = 1 - par

            @pl.when(c + 1 < nchunk)
            def _():
                fire_chunk(c + 1, nxt)

            wait_chunk(par)

            @pl.when(c >= 2)
            def _():
                wait_out(par)

            fire_out(c, par)

    for par in range(2):
        wait_out(par)


def kernel(indices, table):
    b, p = indices.shape
    per_w = b // _NW
    idx = indices.astype(jnp.int32)
    mesh = plsc.VectorSubcoreMesh(core_axis_name="c", subcore_axis_name="s")
    out = pl.kernel(
        _embed_body,
        out_type=jax.ShapeDtypeStruct((b, p, _D), jnp.float32),
        mesh=mesh,
        scratch_types=[
            pltpu.VMEM((per_w, p), jnp.int32),
            pltpu.VMEM((2, _BC, p, _D), jnp.float32),
            pltpu.SemaphoreType.DMA,
            pltpu.SemaphoreType.DMA,
            pltpu.SemaphoreType.DMA,
            pltpu.SemaphoreType.DMA,
        ],
        compiler_params=pltpu.CompilerParams(
            use_tc_tiling_on_sc=True, needs_layout_passes=False
        ),
    )(idx, table)
    return out


# speedup vs baseline: 3.3392x; 1.0540x over previous
import jax
import jax.numpy as jnp
from jax import lax
from jax.experimental import pallas as pl
from jax.experimental.pallas import tpu as pltpu
from jax.experimental.pallas import tpu_sc as plsc

_D = 50
_NC = 2
_NS = 16
_NW = _NC * _NS
_BC = 4


def _embed_body(idx_hbm, table_hbm, out_hbm, idx_v, outb,
                gsem0, gsem1, osem0, osem1):
    per_w, p = idx_v.shape
    cl = _BC * p
    nchunk = per_w // _BC
    wid = lax.axis_index("s") * _NC + lax.axis_index("c")
    base_b = wid * per_w
    pltpu.sync_copy(idx_hbm.at[pl.ds(base_b, per_w)], idx_v)

    gsems = (gsem0, gsem1)
    osems = (osem0, osem1)

    def fire_chunk(c, par):
        for i in range(_BC):
            bb = c * _BC + i
            v0 = idx_v[bb, pl.ds(0, 16)]
            v1 = idx_v[bb, pl.ds(p - 16, 16)]
            for k in range(16):
                pltpu.async_copy(table_hbm.at[v0[k]],
                                 outb.at[par, i, k], gsems[par])
            for k in range(16, p):
                pltpu.async_copy(table_hbm.at[v1[k - (p - 16)]],
                                 outb.at[par, i, k], gsems[par])

    def wait_chunk(par):
        for i in range(_BC):
            for j in range(p):
                pltpu.make_async_copy(
                    table_hbm.at[0], outb.at[par, i, j], gsems[par]).wait()

    def out_slice(c):
        return out_hbm.at[pl.ds(base_b + c * _BC, _BC)]

    def fire_out(c, par):
        pltpu.async_copy(outb.at[par], out_slice(c), osems[par])

    def wait_out(par):
        pltpu.make_async_copy(outb.at[par], out_slice(0), osems[par]).wait()

    fire_chunk(0, 0)

    @pl.loop(0, nchunk // 2)
    def _cc(cc):
        for par in range(2):
            c = cc * 2 + par
            nxt = 1 - par

            @pl.when(c + 1 < nchunk)
            def _():
                fire_chunk(c + 1, nxt)

            wait_chunk(par)

            @pl.when(c >= 2)
            def _():
                wait_out(par)

            fire_out(c, par)

    for par in range(2):
        wait_out(par)


def kernel(indices, table):
    b, p = indices.shape
    per_w = b // _NW
    idx = indices.astype(jnp.int32)
    mesh = plsc.VectorSubcoreMesh(core_axis_name="c", subcore_axis_name="s")
    out = pl.kernel(
        _embed_body,
        out_type=jax.ShapeDtypeStruct((b, p, _D), jnp.float32),
        mesh=mesh,
        scratch_types=[
            pltpu.VMEM((per_w, p), jnp.int32),
            pltpu.VMEM((2, _BC, p, _D), jnp.float32),
            pltpu.SemaphoreType.DMA,
            pltpu.SemaphoreType.DMA,
            pltpu.SemaphoreType.DMA,
            pltpu.SemaphoreType.DMA,
        ],
        compiler_params=pltpu.CompilerParams(
            use_tc_tiling_on_sc=True, needs_layout_passes=False
        ),
    )(idx, table)
    return out
